# baseline (speedup 1.0000x reference)
import functools

import jax
import jax.numpy as jnp
from jax import lax
from jax.experimental import pallas as pl
from jax.experimental.pallas import tpu as pltpu
from jax.experimental.pallas import tpu_sc as plsc

_NBUF = 4


def _make_sc_kernel(N, V, F):
    info = plsc.get_sparse_core_info()
    NC = info.num_cores
    NW = NC * info.num_subcores
    CH = 128
    n_full = N // CH
    tail = N % CH
    per_w = (n_full + NW - 1) // NW
    tail_worker = n_full % NW

    ZR = 3 * N
    ZCH = 256
    z_full = ZR // ZCH
    z_tail = ZR % ZCH
    z_rounds = (z_full + NW - 1) // NW
    z_tail_worker = z_full % NW

    mesh = plsc.VectorSubcoreMesh(core_axis_name="c", subcore_axis_name="s")

    @functools.partial(
        pl.kernel,
        mesh=mesh,
        out_type=[
            jax.ShapeDtypeStruct((N, F), jnp.float32),
            jax.ShapeDtypeStruct((ZR, F), jnp.float32),
        ],
        scratch_types=[
            pltpu.VMEM((per_w, CH), jnp.int32),
            pltpu.VMEM((_NBUF, CH, F), jnp.float32),
            pltpu.VMEM((tail,), jnp.int32),
            pltpu.VMEM((tail, F), jnp.float32),
            pltpu.VMEM((ZCH, F), jnp.float32),
            pltpu.VMEM_SHARED((V, F), jnp.float32),
            pltpu.SemaphoreType.DMA,
            pltpu.SemaphoreType.DMA,
            pltpu.SemaphoreType.DMA,
        ]
        + [pltpu.SemaphoreType.DMA] * (2 * _NBUF),
    )
    def sck(z_hbm, tab_hbm, out_hbm, zout_hbm, idx_v, rows_v, tidx_v,
            trows_v, zbuf, tab_s, sem_i, sem_t, sem_z, *bsems):
        gs, ss = bsems[:_NBUF], bsems[_NBUF:]
        sid = lax.axis_index("s")
        wid = sid * NC + lax.axis_index("c")

        zv = jnp.zeros((16,), jnp.float32)

        def _zrow(r, carry):
            for c in range(F // 16):
                zbuf[r, pl.ds(c * 16, 16)] = zv
            return carry

        lax.fori_loop(0, ZCH, _zrow, 0)

        for g in range(z_rounds):
            k = wid + g * NW

            @pl.when(k < z_full)
            def _(g=g):
                k = wid + g * NW
                pltpu.async_copy(
                    zbuf, zout_hbm.at[pl.ds(k * ZCH, ZCH)], sem_z)

        if z_tail:

            @pl.when(wid == z_tail_worker)
            def _():
                pltpu.async_copy(
                    zbuf.at[pl.ds(0, z_tail)],
                    zout_hbm.at[pl.ds(z_full * ZCH, z_tail)], sem_z)

        @pl.when(sid == 0)
        def _():
            pltpu.sync_copy(tab_hbm, tab_s)

        for i in range(per_w):
            c = wid + i * NW

            @pl.when(c < n_full)
            def _(i=i, c=c):
                pltpu.async_copy(z_hbm.at[pl.ds(c * CH, CH)], idx_v.at[i], sem_i)

        if tail:

            @pl.when(wid == tail_worker)
            def _():
                pltpu.async_copy(
                    z_hbm.at[pl.ds(n_full * CH, tail)], tidx_v, sem_t)

        for i in range(per_w):
            c = wid + i * NW

            @pl.when(c < n_full)
            def _(i=i):
                pltpu.make_async_copy(
                    z_hbm.at[pl.ds(0, CH)], idx_v.at[i], sem_i).wait()

        plsc.subcore_barrier()

        for j in range(per_w + 1):
            if j < per_w:
                b = j % _NBUF
                c = wid + j * NW

                @pl.when(c < n_full)
                def _(j=j, b=b):
                    if j >= _NBUF:
                        pltpu.make_async_copy(
                            rows_v.at[b], out_hbm.at[pl.ds(0, CH)], ss[b]
                        ).wait()
                    pltpu.async_copy(
                        tab_s.at[idx_v.at[j]], rows_v.at[b], gs[b])

            if j >= 1:
                pj = j - 1
                pb = pj % _NBUF
                pc = wid + pj * NW

                @pl.when(pc < n_full)
                def _(pj=pj, pb=pb, pc=pc):
                    pltpu.make_async_copy(
                        tab_s.at[idx_v.at[pj]], rows_v.at[pb], gs[pb]).wait()
                    pltpu.async_copy(
                        rows_v.at[pb], out_hbm.at[pl.ds(pc * CH, CH)], ss[pb])

        for i in range(per_w):
            cond = (wid + i * NW) < n_full
            if i + _NBUF <= per_w - 1:
                cond = jnp.logical_and(
                    cond, jnp.logical_not((wid + (i + _NBUF) * NW) < n_full))

            @pl.when(cond)
            def _(i=i):
                pltpu.make_async_copy(
                    rows_v.at[i % _NBUF], out_hbm.at[pl.ds(0, CH)],
                    ss[i % _NBUF]).wait()

        if tail:

            @pl.when(wid == tail_worker)
            def _():
                pltpu.make_async_copy(
                    z_hbm.at[pl.ds(0, tail)], tidx_v, sem_t).wait()
                pltpu.async_copy(tab_s.at[tidx_v], trows_v, sem_t).wait()
                pltpu.sync_copy(trows_v, out_hbm.at[pl.ds(n_full * CH, tail)])

        for g in range(z_rounds):
            k = wid + g * NW

            @pl.when(k < z_full)
            def _(g=g):
                pltpu.make_async_copy(
                    zbuf, zout_hbm.at[pl.ds(0, ZCH)], sem_z).wait()

        if z_tail:

            @pl.when(wid == z_tail_worker)
            def _():
                pltpu.make_async_copy(
                    zbuf.at[pl.ds(0, z_tail)],
                    zout_hbm.at[pl.ds(0, z_tail)], sem_z).wait()

    return sck


def _make_zeros(N, F):
    B = 5000
    assert N % B == 0
    grid = N // B

    def zk(o5):
        o5[...] = jnp.zeros(o5.shape, jnp.float32)

    return pl.pallas_call(
        zk,
        grid=(grid,),
        out_specs=[pl.BlockSpec((5, B, F), lambda i: (0, i, 0))],
        out_shape=[jax.ShapeDtypeStruct((5, N, F), jnp.float32)],
    )


def kernel(Z, table):
    N = Z.shape[0]
    V, F = table.shape
    x0, z3 = _make_sc_kernel(N, V, F)(Z, table)
    (z5,) = _make_zeros(N, F)()
    return (
        x0.reshape(N, 1, F),
        jnp.transpose(z3.reshape(3, N, F), (1, 0, 2)),
        jnp.transpose(z5, (1, 0, 2)),
    )


# speedup vs baseline: 1.0036x; 1.0036x over previous
import functools

import jax
import jax.numpy as jnp
from jax import lax
from jax.experimental import pallas as pl
from jax.experimental.pallas import tpu as pltpu
from jax.experimental.pallas import tpu_sc as plsc

_NBUF = 3


def _make_sc_kernel(N, V, F):
    info = plsc.get_sparse_core_info()
    NC = info.num_cores
    NW = NC * info.num_subcores
    CH = 128
    n_full = N // CH
    tail = N % CH
    per_w = (n_full + NW - 1) // NW
    tail_worker = n_full % NW

    ZR = 3 * N
    ZCH = 512
    z_full = ZR // ZCH
    z_tail = ZR % ZCH
    z_rounds = (z_full + NW - 1) // NW
    z_tail_worker = z_full % NW

    mesh = plsc.VectorSubcoreMesh(core_axis_name="c", subcore_axis_name="s")

    @functools.partial(
        pl.kernel,
        mesh=mesh,
        out_type=[
            jax.ShapeDtypeStruct((N, F), jnp.float32),
            jax.ShapeDtypeStruct((ZR, F), jnp.float32),
        ],
        scratch_types=[
            pltpu.VMEM((per_w, CH), jnp.int32),
            pltpu.VMEM((_NBUF, CH, F), jnp.float32),
            pltpu.VMEM((tail,), jnp.int32),
            pltpu.VMEM((tail, F), jnp.float32),
            pltpu.VMEM((ZCH, F), jnp.float32),
            pltpu.VMEM_SHARED((V, F), jnp.float32),
            pltpu.SemaphoreType.DMA,
            pltpu.SemaphoreType.DMA,
            pltpu.SemaphoreType.DMA,
        ]
        + [pltpu.SemaphoreType.DMA] * (2 * _NBUF),
    )
    def sck(z_hbm, tab_hbm, out_hbm, zout_hbm, idx_v, rows_v, tidx_v,
            trows_v, zbuf, tab_s, sem_i, sem_t, sem_z, *bsems):
        gs, ss = bsems[:_NBUF], bsems[_NBUF:]
        sid = lax.axis_index("s")
        wid = sid * NC + lax.axis_index("c")

        zv = jnp.zeros((16,), jnp.float32)

        def _zrow(r, carry):
            for c in range(F // 16):
                zbuf[r, pl.ds(c * 16, 16)] = zv
            return carry

        lax.fori_loop(0, ZCH, _zrow, 0)

        def _zfire(g, carry):
            k = wid + g * NW

            @pl.when(k < z_full)
            def _():
                pltpu.async_copy(
                    zbuf, zout_hbm.at[pl.ds(k * ZCH, ZCH)], sem_z)

            return carry

        lax.fori_loop(0, z_rounds, _zfire, 0)

        if z_tail:

            @pl.when(wid == z_tail_worker)
            def _():
                pltpu.async_copy(
                    zbuf.at[pl.ds(0, z_tail)],
                    zout_hbm.at[pl.ds(z_full * ZCH, z_tail)], sem_z)

        @pl.when(sid == 0)
        def _():
            pltpu.sync_copy(tab_hbm, tab_s)

        for i in range(per_w):
            c = wid + i * NW

            @pl.when(c < n_full)
            def _(i=i, c=c):
                pltpu.async_copy(z_hbm.at[pl.ds(c * CH, CH)], idx_v.at[i], sem_i)

        if tail:

            @pl.when(wid == tail_worker)
            def _():
                pltpu.async_copy(
                    z_hbm.at[pl.ds(n_full * CH, tail)], tidx_v, sem_t)

        for i in range(per_w):
            c = wid + i * NW

            @pl.when(c < n_full)
            def _(i=i):
                pltpu.make_async_copy(
                    z_hbm.at[pl.ds(0, CH)], idx_v.at[i], sem_i).wait()

        plsc.subcore_barrier()

        for j in range(per_w + 1):
            if j < per_w:
                b = j % _NBUF
                c = wid + j * NW

                @pl.when(c < n_full)
                def _(j=j, b=b):
                    if j >= _NBUF:
                        pltpu.make_async_copy(
                            rows_v.at[b], out_hbm.at[pl.ds(0, CH)], ss[b]
                        ).wait()
                    pltpu.async_copy(
                        tab_s.at[idx_v.at[j]], rows_v.at[b], gs[b])

            if j >= 1:
                pj = j - 1
                pb = pj % _NBUF
                pc = wid + pj * NW

                @pl.when(pc < n_full)
                def _(pj=pj, pb=pb, pc=pc):
                    pltpu.make_async_copy(
                        tab_s.at[idx_v.at[pj]], rows_v.at[pb], gs[pb]).wait()
                    pltpu.async_copy(
                        rows_v.at[pb], out_hbm.at[pl.ds(pc * CH, CH)], ss[pb])

        for i in range(per_w):
            cond = (wid + i * NW) < n_full
            if i + _NBUF <= per_w - 1:
                cond = jnp.logical_and(
                    cond, jnp.logical_not((wid + (i + _NBUF) * NW) < n_full))

            @pl.when(cond)
            def _(i=i):
                pltpu.make_async_copy(
                    rows_v.at[i % _NBUF], out_hbm.at[pl.ds(0, CH)],
                    ss[i % _NBUF]).wait()

        if tail:

            @pl.when(wid == tail_worker)
            def _():
                pltpu.make_async_copy(
                    z_hbm.at[pl.ds(0, tail)], tidx_v, sem_t).wait()
                pltpu.async_copy(tab_s.at[tidx_v], trows_v, sem_t).wait()
                pltpu.sync_copy(trows_v, out_hbm.at[pl.ds(n_full * CH, tail)])

        def _zdrain(g, carry):
            k = wid + g * NW

            @pl.when(k < z_full)
            def _():
                pltpu.make_async_copy(
                    zbuf, zout_hbm.at[pl.ds(0, ZCH)], sem_z).wait()

            return carry

        lax.fori_loop(0, z_rounds, _zdrain, 0)

        if z_tail:

            @pl.when(wid == z_tail_worker)
            def _():
                pltpu.make_async_copy(
                    zbuf.at[pl.ds(0, z_tail)],
                    zout_hbm.at[pl.ds(0, z_tail)], sem_z).wait()

    return sck


def _make_zeros(N, F):
    B = 2000
    assert N % B == 0
    grid = N // B

    def zk(o5):
        o5[...] = jnp.zeros(o5.shape, jnp.float32)

    return pl.pallas_call(
        zk,
        grid=(grid,),
        out_specs=[pl.BlockSpec((5, B, F), lambda i: (0, i, 0))],
        out_shape=[jax.ShapeDtypeStruct((5, N, F), jnp.float32)],
    )


def kernel(Z, table):
    N = Z.shape[0]
    V, F = table.shape
    x0, z3 = _make_sc_kernel(N, V, F)(Z, table)
    (z5,) = _make_zeros(N, F)()
    return (
        x0.reshape(N, 1, F),
        jnp.transpose(z3.reshape(3, N, F), (1, 0, 2)),
        jnp.transpose(z5, (1, 0, 2)),
    )


# speedup vs baseline: 1.0070x; 1.0034x over previous
import functools

import jax
import jax.numpy as jnp
from jax import lax
from jax.experimental import pallas as pl
from jax.experimental.pallas import tpu as pltpu
from jax.experimental.pallas import tpu_sc as plsc

_NBUF = 4


def _make_sc_kernel(N, V, F):
    info = plsc.get_sparse_core_info()
    NC = info.num_cores
    NW = NC * info.num_subcores
    CH = 128
    n_full = N // CH
    tail = N % CH
    per_w = (n_full + NW - 1) // NW
    tail_worker = n_full % NW

    ZR = 3 * N
    ZCH = 256
    z_full = ZR // ZCH
    z_tail = ZR % ZCH
    z_rounds = (z_full + NW - 1) // NW
    z_tail_worker = z_full % NW

    mesh = plsc.VectorSubcoreMesh(core_axis_name="c", subcore_axis_name="s")

    @functools.partial(
        pl.kernel,
        mesh=mesh,
        out_type=[
            jax.ShapeDtypeStruct((N, F), jnp.float32),
            jax.ShapeDtypeStruct((ZR, F), jnp.float32),
        ],
        scratch_types=[
            pltpu.VMEM((per_w, CH), jnp.int32),
            pltpu.VMEM((_NBUF, CH, F), jnp.float32),
            pltpu.VMEM((tail,), jnp.int32),
            pltpu.VMEM((tail, F), jnp.float32),
            pltpu.VMEM((ZCH, F), jnp.float32),
            pltpu.VMEM_SHARED((V, F), jnp.float32),
            pltpu.SemaphoreType.DMA,
            pltpu.SemaphoreType.DMA,
            pltpu.SemaphoreType.DMA,
        ]
        + [pltpu.SemaphoreType.DMA] * (2 * _NBUF),
    )
    def sck(z_hbm, tab_hbm, out_hbm, zout_hbm, idx_v, rows_v, tidx_v,
            trows_v, zbuf, tab_s, sem_i, sem_t, sem_z, *bsems):
        gs, ss = bsems[:_NBUF], bsems[_NBUF:]
        sid = lax.axis_index("s")
        wid = sid * NC + lax.axis_index("c")

        zv = jnp.zeros((16,), jnp.float32)

        def _zrow(r, carry):
            for c in range(F // 16):
                zbuf[r, pl.ds(c * 16, 16)] = zv
            return carry

        lax.fori_loop(0, ZCH, _zrow, 0)

        for g in range(z_rounds):
            k = wid + g * NW

            @pl.when(k < z_full)
            def _(g=g):
                k = wid + g * NW
                pltpu.async_copy(
                    zbuf, zout_hbm.at[pl.ds(k * ZCH, ZCH)], sem_z)

        if z_tail:

            @pl.when(wid == z_tail_worker)
            def _():
                pltpu.async_copy(
                    zbuf.at[pl.ds(0, z_tail)],
                    zout_hbm.at[pl.ds(z_full * ZCH, z_tail)], sem_z)

        @pl.when(sid == 0)
        def _():
            pltpu.sync_copy(tab_hbm, tab_s)

        for i in range(per_w):
            c = wid + i * NW

            @pl.when(c < n_full)
            def _(i=i, c=c):
                pltpu.async_copy(z_hbm.at[pl.ds(c * CH, CH)], idx_v.at[i], sem_i)

        if tail:

            @pl.when(wid == tail_worker)
            def _():
                pltpu.async_copy(
                    z_hbm.at[pl.ds(n_full * CH, tail)], tidx_v, sem_t)

        for i in range(per_w):
            c = wid + i * NW

            @pl.when(c < n_full)
            def _(i=i):
                pltpu.make_async_copy(
                    z_hbm.at[pl.ds(0, CH)], idx_v.at[i], sem_i).wait()

        plsc.subcore_barrier()

        for j in range(per_w + 1):
            if j < per_w:
                b = j % _NBUF
                c = wid + j * NW

                @pl.when(c < n_full)
                def _(j=j, b=b):
                    if j >= _NBUF:
                        pltpu.make_async_copy(
                            rows_v.at[b], out_hbm.at[pl.ds(0, CH)], ss[b]
                        ).wait()
                    pltpu.async_copy(
                        tab_s.at[idx_v.at[j]], rows_v.at[b], gs[b])

            if j >= 1:
                pj = j - 1
                pb = pj % _NBUF
                pc = wid + pj * NW

                @pl.when(pc < n_full)
                def _(pj=pj, pb=pb, pc=pc):
                    pltpu.make_async_copy(
                        tab_s.at[idx_v.at[pj]], rows_v.at[pb], gs[pb]).wait()
                    pltpu.async_copy(
                        rows_v.at[pb], out_hbm.at[pl.ds(pc * CH, CH)], ss[pb])

        for i in range(per_w):
            cond = (wid + i * NW) < n_full
            if i + _NBUF <= per_w - 1:
                cond = jnp.logical_and(
                    cond, jnp.logical_not((wid + (i + _NBUF) * NW) < n_full))

            @pl.when(cond)
            def _(i=i):
                pltpu.make_async_copy(
                    rows_v.at[i % _NBUF], out_hbm.at[pl.ds(0, CH)],
                    ss[i % _NBUF]).wait()

        if tail:

            @pl.when(wid == tail_worker)
            def _():
                pltpu.make_async_copy(
                    z_hbm.at[pl.ds(0, tail)], tidx_v, sem_t).wait()
                pltpu.async_copy(tab_s.at[tidx_v], trows_v, sem_t).wait()
                pltpu.sync_copy(trows_v, out_hbm.at[pl.ds(n_full * CH, tail)])

        for g in range(z_rounds):
            k = wid + g * NW

            @pl.when(k < z_full)
            def _(g=g):
                pltpu.make_async_copy(
                    zbuf, zout_hbm.at[pl.ds(0, ZCH)], sem_z).wait()

        if z_tail:

            @pl.when(wid == z_tail_worker)
            def _():
                pltpu.make_async_copy(
                    zbuf.at[pl.ds(0, z_tail)],
                    zout_hbm.at[pl.ds(0, z_tail)], sem_z).wait()

    return sck


def _make_zeros(N, F):
    B = 1000
    assert N % B == 0
    grid = N // B

    def zk(o5):
        o5[...] = jnp.zeros(o5.shape, jnp.float32)

    return pl.pallas_call(
        zk,
        grid=(grid,),
        out_specs=[pl.BlockSpec((5, B, F), lambda i: (0, i, 0))],
        out_shape=[jax.ShapeDtypeStruct((5, N, F), jnp.float32)],
    )


def kernel(Z, table):
    N = Z.shape[0]
    V, F = table.shape
    x0, z3 = _make_sc_kernel(N, V, F)(Z, table)
    (z5,) = _make_zeros(N, F)()
    return (
        x0.reshape(N, 1, F),
        jnp.transpose(z3.reshape(3, N, F), (1, 0, 2)),
        jnp.transpose(z5, (1, 0, 2)),
    )


# speedup vs baseline: 1.0082x; 1.0012x over previous
import functools

import jax
import jax.numpy as jnp
from jax import lax
from jax.experimental import pallas as pl
from jax.experimental.pallas import tpu as pltpu
from jax.experimental.pallas import tpu_sc as plsc

_NBUF = 4


def _make_sc_kernel(N, V, F):
    info = plsc.get_sparse_core_info()
    NC = info.num_cores
    NW = NC * info.num_subcores
    CH = 128
    n_full = N // CH
    tail = N % CH
    per_w = (n_full + NW - 1) // NW
    tail_worker = n_full % NW

    ZR = 3 * N
    ZCH = 256
    z_full = ZR // ZCH
    z_tail = ZR % ZCH
    z_rounds = (z_full + NW - 1) // NW
    z_tail_worker = z_full % NW

    mesh = plsc.VectorSubcoreMesh(core_axis_name="c", subcore_axis_name="s")

    @functools.partial(
        pl.kernel,
        mesh=mesh,
        out_type=[
            jax.ShapeDtypeStruct((N, F), jnp.float32),
            jax.ShapeDtypeStruct((ZR, F), jnp.float32),
        ],
        scratch_types=[
            pltpu.VMEM((per_w, CH), jnp.int32),
            pltpu.VMEM((_NBUF, CH, F), jnp.float32),
            pltpu.VMEM((tail,), jnp.int32),
            pltpu.VMEM((tail, F), jnp.float32),
            pltpu.VMEM((ZCH, F), jnp.float32),
            pltpu.VMEM_SHARED((V, F), jnp.float32),
            pltpu.SemaphoreType.DMA,
            pltpu.SemaphoreType.DMA,
            pltpu.SemaphoreType.DMA,
        ]
        + [pltpu.SemaphoreType.DMA] * (2 * _NBUF),
    )
    def sck(z_hbm, tab_hbm, out_hbm, zout_hbm, idx_v, rows_v, tidx_v,
            trows_v, zbuf, tab_s, sem_i, sem_t, sem_z, *bsems):
        gs, ss = bsems[:_NBUF], bsems[_NBUF:]
        sid = lax.axis_index("s")
        wid = sid * NC + lax.axis_index("c")

        zv = jnp.zeros((16,), jnp.float32)

        def _zrow(r, carry):
            for c in range(F // 16):
                zbuf[r, pl.ds(c * 16, 16)] = zv
            return carry

        lax.fori_loop(0, ZCH, _zrow, 0)

        for g in range(z_rounds):
            k = wid + g * NW

            @pl.when(k < z_full)
            def _(g=g):
                k = wid + g * NW
                pltpu.async_copy(
                    zbuf, zout_hbm.at[pl.ds(k * ZCH, ZCH)], sem_z)

        if z_tail:

            @pl.when(wid == z_tail_worker)
            def _():
                pltpu.async_copy(
                    zbuf.at[pl.ds(0, z_tail)],
                    zout_hbm.at[pl.ds(z_full * ZCH, z_tail)], sem_z)

        @pl.when(sid == 0)
        def _():
            pltpu.sync_copy(tab_hbm, tab_s)

        for i in range(per_w):
            c = wid + i * NW

            @pl.when(c < n_full)
            def _(i=i, c=c):
                pltpu.async_copy(z_hbm.at[pl.ds(c * CH, CH)], idx_v.at[i], sem_i)

        if tail:

            @pl.when(wid == tail_worker)
            def _():
                pltpu.async_copy(
                    z_hbm.at[pl.ds(n_full * CH, tail)], tidx_v, sem_t)

        for i in range(per_w):
            c = wid + i * NW

            @pl.when(c < n_full)
            def _(i=i):
                pltpu.make_async_copy(
                    z_hbm.at[pl.ds(0, CH)], idx_v.at[i], sem_i).wait()

        plsc.subcore_barrier()

        for j in range(per_w + 1):
            if j < per_w:
                b = j % _NBUF
                c = wid + j * NW

                @pl.when(c < n_full)
                def _(j=j, b=b):
                    if j >= _NBUF:
                        pltpu.make_async_copy(
                            rows_v.at[b], out_hbm.at[pl.ds(0, CH)], ss[b]
                        ).wait()
                    pltpu.async_copy(
                        tab_s.at[idx_v.at[j]], rows_v.at[b], gs[b])

            if j >= 1:
                pj = j - 1
                pb = pj % _NBUF
                pc = wid + pj * NW

                @pl.when(pc < n_full)
                def _(pj=pj, pb=pb, pc=pc):
                    pltpu.make_async_copy(
                        tab_s.at[idx_v.at[pj]], rows_v.at[pb], gs[pb]).wait()
                    pltpu.async_copy(
                        rows_v.at[pb], out_hbm.at[pl.ds(pc * CH, CH)], ss[pb])

        for i in range(per_w):
            cond = (wid + i * NW) < n_full
            if i + _NBUF <= per_w - 1:
                cond = jnp.logical_and(
                    cond, jnp.logical_not((wid + (i + _NBUF) * NW) < n_full))

            @pl.when(cond)
            def _(i=i):
                pltpu.make_async_copy(
                    rows_v.at[i % _NBUF], out_hbm.at[pl.ds(0, CH)],
                    ss[i % _NBUF]).wait()

        if tail:

            @pl.when(wid == tail_worker)
            def _():
                pltpu.make_async_copy(
                    z_hbm.at[pl.ds(0, tail)], tidx_v, sem_t).wait()
                pltpu.async_copy(tab_s.at[tidx_v], trows_v, sem_t).wait()
                pltpu.sync_copy(trows_v, out_hbm.at[pl.ds(n_full * CH, tail)])

        for g in range(z_rounds):
            k = wid + g * NW

            @pl.when(k < z_full)
            def _(g=g):
                pltpu.make_async_copy(
                    zbuf, zout_hbm.at[pl.ds(0, ZCH)], sem_z).wait()

        if z_tail:

            @pl.when(wid == z_tail_worker)
            def _():
                pltpu.make_async_copy(
                    zbuf.at[pl.ds(0, z_tail)],
                    zout_hbm.at[pl.ds(0, z_tail)], sem_z).wait()

    return sck


def _make_zeros(N, F):
    B = 2000
    assert N % B == 0
    grid = N // B

    def zk(o5):
        o5[...] = jnp.zeros(o5.shape, jnp.float32)

    return pl.pallas_call(
        zk,
        grid=(grid,),
        out_specs=[pl.BlockSpec((5, B, F), lambda i: (0, i, 0))],
        out_shape=[jax.ShapeDtypeStruct((5, N, F), jnp.float32)],
    )


def kernel(Z, table):
    N = Z.shape[0]
    V, F = table.shape
    x0, z3 = _make_sc_kernel(N, V, F)(Z, table)
    (z5,) = _make_zeros(N, F)()
    return (
        x0.reshape(N, 1, F),
        jnp.transpose(z3.reshape(3, N, F), (1, 0, 2)),
        jnp.transpose(z5, (1, 0, 2)),
    )
